# baseline (speedup 1.0000x reference)
import functools

import jax
import jax.numpy as jnp
from jax import lax
from jax.experimental import pallas as pl
from jax.experimental.pallas import tpu as pltpu
from jax.experimental.pallas import tpu_sc as plsc

B, L, D = 16, 4096, 1024
NC = 2
C = D // NC
T = 64
NV = C // 16

_mesh = plsc.VectorSubcoreMesh(core_axis_name="c", subcore_axis_name="s")


@functools.partial(
    pl.kernel,
    mesh=_mesh,
    out_type=jax.ShapeDtypeStruct((B, D), jnp.float32),
    scratch_types=[
        pltpu.VMEM((32,), jnp.int32),
        pltpu.VMEM((32,), jnp.float32),
        pltpu.VMEM((T, C), jnp.float32),
        pltpu.VMEM((C,), jnp.float32),
    ],
)
def _ragged_mean_sc(emb_hbm, len_hbm, inv_hbm, out_hbm,
                    len_v, inv_v, buf, outb):
    b = lax.axis_index("s")
    col0 = lax.axis_index("c") * C

    pltpu.sync_copy(len_hbm, len_v.at[pl.ds(0, 16)])
    pltpu.sync_copy(inv_hbm, inv_v.at[pl.ds(0, 16)])

    len_b = len_v[pl.ds(b, 16)][0]
    inv_b = inv_v[pl.ds(b, 16)][0]

    nchunks = (len_b + (T - 1)) // T

    def chunk_body(k, accs):
        t0 = k * T
        pltpu.sync_copy(emb_hbm.at[b, pl.ds(t0, T), pl.ds(col0, C)], buf)
        nvalid = jnp.minimum(T, len_b - t0)

        def tok_body(j, a):
            return tuple(a[v] + buf[j, pl.ds(v * 16, 16)] for v in range(NV))

        return lax.fori_loop(0, nvalid, tok_body, accs)

    zero = jnp.zeros((16,), jnp.float32)
    accs = lax.fori_loop(0, nchunks, chunk_body, (zero,) * NV)

    for v in range(NV):
        outb[pl.ds(v * 16, 16)] = accs[v] * inv_b
    pltpu.sync_copy(outb, out_hbm.at[b, pl.ds(col0, C)])


def kernel(embeddings, lengths):
    lengths_i = lengths.astype(jnp.int32)
    inv = 1.0 / lengths_i.astype(jnp.float32)
    return _ragged_mean_sc(embeddings, lengths_i, inv)
